# baseline (speedup 1.0000x reference)
import functools

import jax
import jax.numpy as jnp
from jax import lax
from jax.experimental import pallas as pl
from jax.experimental.pallas import tpu as pltpu
from jax.experimental.pallas import tpu_sc as plsc

NUM_FEATURES = 9
NUM_BINS = 101
EMBED_DIM = 512
BATCH = 16384

_ROWS = BATCH * NUM_FEATURES
_NW = 32
_ROWS_PER_W = _ROWS // _NW
_CHUNK = 128
_NCHUNK = _ROWS_PER_W // _CHUNK
_LANES = 16
_MAGIC = 8388608.0


def _sc_gather(feat_flat, w_flat):
    mesh = plsc.VectorSubcoreMesh(core_axis_name="c", subcore_axis_name="s")

    @functools.partial(
        pl.kernel,
        mesh=mesh,
        out_type=jax.ShapeDtypeStruct((_ROWS, EMBED_DIM), jnp.float32),
        scratch_types=[
            pltpu.VMEM((_CHUNK,), jnp.float32),
            pltpu.VMEM((_CHUNK,), jnp.int32),
            pltpu.VMEM((_CHUNK, EMBED_DIM), jnp.float32),
            pltpu.SemaphoreType.DMA,
        ],
    )
    def body(feat_hbm, w_hbm, out_hbm, feat_v, idx_v, rows_v, sem):
        wid = lax.axis_index("s") * 2 + lax.axis_index("c")
        w_base = wid * _ROWS_PER_W
        iota = lax.iota(jnp.int32, _LANES)

        def chunk(j, carry):
            base = w_base + j * _CHUNK
            pltpu.sync_copy(feat_hbm.at[pl.ds(base, _CHUNK)], feat_v)
            for i in range(_CHUNK // _LANES):
                x = feat_v[pl.ds(i * _LANES, _LANES)]
                xc = jnp.minimum(jnp.maximum(x, 0.0), float(NUM_BINS - 1))
                r = (xc + _MAGIC) - _MAGIC
                fid = lax.rem(base + i * _LANES + iota,
                              jnp.int32(NUM_FEATURES))
                idx_v[pl.ds(i * _LANES, _LANES)] = (
                    fid * NUM_BINS + r.astype(jnp.int32))
            pltpu.async_copy(w_hbm.at[idx_v], rows_v, sem).wait()
            pltpu.sync_copy(rows_v, out_hbm.at[pl.ds(base, _CHUNK)])
            return carry

        lax.fori_loop(0, _NCHUNK, chunk, 0)

    return body(feat_flat, w_flat)


def kernel(features, W):
    feat_flat = features.reshape(_ROWS)
    w_flat = W.reshape(NUM_FEATURES * NUM_BINS, EMBED_DIM)
    out = _sc_gather(feat_flat, w_flat)
    return out.reshape(BATCH, NUM_FEATURES * EMBED_DIM)
